# baseline (speedup 1.0000x reference)
import math

import jax
import jax.numpy as jnp
from jax import lax
from jax.experimental import pallas as pl
from jax.experimental.pallas import tpu as pltpu
from jax.experimental.pallas import tpu_sc as plsc

EMB_D = 128
PACK_D = EMB_D // 2
SCALE = math.sqrt(float(EMB_D))

NC = 2
NS = 16
NW = NC * NS

CH = 128
PAIR = 2


def _make_gather(nch):
    npair = nch // PAIR
    rows = PAIR * CH
    mesh = plsc.VectorSubcoreMesh(
        core_axis_name="c", subcore_axis_name="s",
        num_cores=NC, num_subcores=NS,
    )

    def body(table_hbm, tok_hbm, out_hbm, idx_v, pb0, pb1, fb0, fb1,
             gs0, gs1, ws0, ws1):
        pbufs = (pb0, pb1)
        fbufs = (fb0, fb1)
        gsems = (gs0, gs1)
        wsems = (ws0, ws1)
        wid = lax.axis_index("s") * NC + lax.axis_index("c")
        pltpu.sync_copy(tok_hbm.at[wid], idx_v)

        def fire(p, s):
            for j in range(PAIR):
                pltpu.async_copy(
                    table_hbm.at[idx_v.at[p * PAIR + j]],
                    pbufs[s].at[pl.ds(j * CH, CH)], gsems[s])

        def drain_g(s):
            pltpu.make_async_copy(
                table_hbm.at[pl.ds(0, rows)], pbufs[s], gsems[s]).wait()

        def drain_w(s):
            pltpu.make_async_copy(
                fbufs[s], out_hbm.at[wid, 0], wsems[s]).wait()

        fire(0, 0)

        @pl.loop(0, npair, step=2)
        def _pass(g):
            for b in range(2):
                p = g + b

                @pl.when(p + 1 < npair)
                def _():
                    fire(p + 1, 1 - b)

                drain_g(b)

                @pl.when(p >= 2)
                def _():
                    drain_w(b)

                pbuf, fbuf = pbufs[b], fbufs[b]

                @pl.loop(0, rows, unroll=2)
                def _decode_row(r):
                    for k in range(PACK_D // 16):
                        v = pbuf[r, pl.ds(k * 16, 16)]
                        lo = plsc.bitcast(v << 16, jnp.float32)
                        hi = plsc.bitcast(v & -65536, jnp.float32)
                        fbuf[r, pl.ds(k * 32, 16)] = lo * SCALE
                        fbuf[r, pl.ds(k * 32 + 16, 16)] = hi * SCALE

                pltpu.async_copy(fbuf, out_hbm.at[wid, p], wsems[b])

        drain_w(0)
        drain_w(1)

    return pl.kernel(
        body,
        out_type=jax.ShapeDtypeStruct((NW, npair, rows, EMB_D), jnp.float32),
        mesh=mesh,
        compiler_params=pltpu.CompilerParams(
            needs_layout_passes=False, use_tc_tiling_on_sc=False),
        scratch_types=[
            pltpu.VMEM((nch, CH), jnp.int32),
            *[pltpu.VMEM((rows, PACK_D), jnp.int32) for _ in range(2)],
            *[pltpu.VMEM((rows, EMB_D), jnp.float32) for _ in range(2)],
            *[pltpu.SemaphoreType.DMA for _ in range(4)],
        ],
    )


def kernel(tokens, embedding_weight):
    batch, seq = tokens.shape
    total = batch * seq
    assert total % (NW * CH) == 0
    nch = total // (NW * CH)
    v, d = embedding_weight.shape
    assert d == EMB_D

    w16 = embedding_weight.astype(jnp.bfloat16)
    wp = jax.lax.bitcast_convert_type(
        w16.reshape(v, d // 32, 2, 16).swapaxes(2, 3), jnp.int32)
    wp = wp.reshape(v, PACK_D)

    tok = tokens.reshape(NW, nch, CH).astype(jnp.int32)
    out = _make_gather(nch)(wp, tok)
    return out.reshape(batch, seq, EMB_D)


# speedup vs baseline: 2.3989x; 2.3989x over previous
import math

import jax
import jax.numpy as jnp
from jax import lax
from jax.experimental import pallas as pl
from jax.experimental.pallas import tpu as pltpu
from jax.experimental.pallas import tpu_sc as plsc

EMB_D = 128
SCALE = math.sqrt(float(EMB_D))

NC = 2
NS = 16
NW = NC * NS

CH = 128


def _scale_body(w_ref, o_ref):
    o_ref[...] = w_ref[...] * SCALE


def _scale_table(w):
    v, d = w.shape
    br = 2000
    assert v % br == 0
    return pl.pallas_call(
        _scale_body,
        grid=(v // br,),
        in_specs=[pl.BlockSpec((br, d), lambda i: (i, 0))],
        out_specs=pl.BlockSpec((br, d), lambda i: (i, 0)),
        out_shape=jax.ShapeDtypeStruct((v, d), w.dtype),
    )(w)


PAIR = 2


def _make_gather(nch):
    npair = nch // PAIR
    mesh = plsc.VectorSubcoreMesh(
        core_axis_name="c", subcore_axis_name="s",
        num_cores=NC, num_subcores=NS,
    )

    def body(table_hbm, tok_hbm, out_hbm, idx_v, buf0, buf1, gs0, gs1,
             ws0, ws1):
        bufs = (buf0, buf1)
        gsems = (gs0, gs1)
        wsems = (ws0, ws1)
        wid = lax.axis_index("s") * NC + lax.axis_index("c")
        pltpu.sync_copy(tok_hbm.at[wid], idx_v)

        def fire(p, s):
            for j in range(PAIR):
                pltpu.async_copy(
                    table_hbm.at[idx_v.at[p * PAIR + j]],
                    bufs[s].at[pl.ds(j * CH, CH)], gsems[s])

        def drain_g(s):
            pltpu.make_async_copy(out_hbm.at[wid, 0], bufs[s], gsems[s]).wait()

        def drain_w(s):
            pltpu.make_async_copy(bufs[s], out_hbm.at[wid, 0], wsems[s]).wait()

        fire(0, 0)

        @pl.loop(0, npair, step=2)
        def _pass(g):
            for b in range(2):
                p = g + b

                @pl.when(p + 1 < npair)
                def _():
                    @pl.when(p >= 1)
                    def _():
                        drain_w(1 - b)
                    fire(p + 1, 1 - b)

                drain_g(b)
                buf = bufs[b]

                @pl.loop(0, PAIR * CH, unroll=4)
                def _scale_row(r):
                    for k in range(EMB_D // 16):
                        sl = pl.ds(k * 16, 16)
                        buf[r, sl] = buf[r, sl] * SCALE

                pltpu.async_copy(bufs[b], out_hbm.at[wid, p], wsems[b])

        drain_w(0)
        drain_w(1)

    return pl.kernel(
        body,
        out_type=jax.ShapeDtypeStruct(
            (NW, npair, PAIR * CH, EMB_D), jnp.float32),
        mesh=mesh,
        scratch_types=[
            pltpu.VMEM((nch, CH), jnp.int32),
            *[pltpu.VMEM((PAIR * CH, EMB_D), jnp.float32) for _ in range(2)],
            *[pltpu.SemaphoreType.DMA for _ in range(4)],
        ],
    )


def kernel(tokens, embedding_weight):
    batch, seq = tokens.shape
    total = batch * seq
    assert total % (NW * CH) == 0
    nch = total // (NW * CH)

    tok = tokens.reshape(NW, nch, CH).astype(jnp.int32)
    out = _make_gather(nch)(embedding_weight, tok)
    return out.reshape(batch, seq, EMB_D)


# speedup vs baseline: 2.4051x; 1.0026x over previous
import math

import jax
import jax.numpy as jnp
from jax import lax
from jax.experimental import pallas as pl
from jax.experimental.pallas import tpu as pltpu
from jax.experimental.pallas import tpu_sc as plsc

EMB_D = 128
SCALE = math.sqrt(float(EMB_D))

NC = 2
NS = 16
NW = NC * NS

CH = 128


PAIR = 2


def _make_gather(nch):
    npair = nch // PAIR
    mesh = plsc.VectorSubcoreMesh(
        core_axis_name="c", subcore_axis_name="s",
        num_cores=NC, num_subcores=NS,
    )

    def body(table_hbm, tok_hbm, out_hbm, idx_v, buf0, buf1, gs0, gs1,
             ws0, ws1):
        bufs = (buf0, buf1)
        gsems = (gs0, gs1)
        wsems = (ws0, ws1)
        wid = lax.axis_index("s") * NC + lax.axis_index("c")
        pltpu.sync_copy(tok_hbm.at[wid], idx_v)

        def fire(p, s):
            for j in range(PAIR):
                pltpu.async_copy(
                    table_hbm.at[idx_v.at[p * PAIR + j]],
                    bufs[s].at[pl.ds(j * CH, CH)], gsems[s])

        def drain_g(s):
            pltpu.make_async_copy(out_hbm.at[wid, 0], bufs[s], gsems[s]).wait()

        def drain_w(s):
            pltpu.make_async_copy(bufs[s], out_hbm.at[wid, 0], wsems[s]).wait()

        fire(0, 0)

        @pl.loop(0, npair, step=2)
        def _pass(g):
            for b in range(2):
                p = g + b

                @pl.when(p + 1 < npair)
                def _():
                    @pl.when(p >= 1)
                    def _():
                        drain_w(1 - b)
                    fire(p + 1, 1 - b)

                drain_g(b)
                buf = bufs[b]

                @pl.loop(0, PAIR * CH, unroll=4)
                def _scale_row(r):
                    for k in range(EMB_D // 16):
                        sl = pl.ds(k * 16, 16)
                        buf[r, sl] = buf[r, sl] * SCALE

                pltpu.async_copy(bufs[b], out_hbm.at[wid, p], wsems[b])

        drain_w(0)
        drain_w(1)

    return pl.kernel(
        body,
        out_type=jax.ShapeDtypeStruct(
            (NW, npair, PAIR * CH, EMB_D), jnp.float32),
        mesh=mesh,
        scratch_types=[
            pltpu.VMEM((nch, CH), jnp.int32),
            *[pltpu.VMEM((PAIR * CH, EMB_D), jnp.float32) for _ in range(2)],
            *[pltpu.SemaphoreType.DMA for _ in range(4)],
        ],
    )


def kernel(tokens, embedding_weight):
    batch, seq = tokens.shape
    total = batch * seq
    assert total % (NW * CH) == 0
    nch = total // (NW * CH)

    tok = tokens.reshape(NW, nch, CH).astype(jnp.int32)
    out = _make_gather(nch)(embedding_weight, tok)
    return out.reshape(batch, seq, EMB_D)
